# baseline (speedup 1.0000x reference)
import functools

import jax
import jax.numpy as jnp
from jax import lax
from jax.experimental import pallas as pl
from jax.experimental.pallas import tpu as pltpu
from jax.experimental.pallas import tpu_sc as plsc

B, CDD, HIS, K = 16, 5, 100, 5
SIG, LVL, HID = 32, 3, 256
THRESHOLD = 0.1
ROW = SIG * LVL * HID
NROWS = B * CDD * K
NC, NS = 2, 16
NW = NC * NS
CHUNKS = ROW // 16
RPW = -(-NROWS // NW)


def _select_body(cdd_ref, his_ref, wt_ref, b_ref, gidx_ref, vals_ref):
    bidx = pl.program_id(0)
    cdd = cdd_ref[0]
    his = his_ref[0]
    wt = wt_ref[...]
    bias = b_ref[...]
    dn = (((1,), (0,)), ((), ()))
    cp = lax.dot_general(cdd, wt, dn, preferred_element_type=jnp.float32) + bias
    hp = lax.dot_general(his, wt, dn, preferred_element_type=jnp.float32) + bias
    cn = jnp.sqrt(jnp.sum(cp * cp, axis=1, keepdims=True))
    hn = jnp.sqrt(jnp.sum(hp * hp, axis=1, keepdims=True))
    cp = cp / jnp.maximum(cn, 1e-12)
    hp = hp / jnp.maximum(hn, 1e-12)
    attn = lax.dot_general(cp, hp, (((1,), (1,)), ((), ())),
                           preferred_element_type=jnp.float32)
    cols = lax.broadcasted_iota(jnp.int32, (CDD, HIS), 1)
    a = attn
    vs, ids = [], []
    for _ in range(K):
        m = jnp.max(a, axis=1, keepdims=True)
        amax = jnp.min(jnp.where(a == m, cols, HIS), axis=1, keepdims=True)
        vs.append(m)
        ids.append(amax)
        a = jnp.where(cols == amax, -jnp.inf, a)
    vals = jnp.concatenate(vs, axis=1)
    idx = jnp.concatenate(ids, axis=1)
    masked = jnp.where(vals < THRESHOLD, 0.0, vals)
    gidx_ref[0] = idx + bidx * HIS
    vals_ref[0] = masked


def _select(cdd_repr, his_repr, sel_Wt, sel_b2):
    return pl.pallas_call(
        _select_body,
        grid=(B,),
        in_specs=[
            pl.BlockSpec((1, CDD, HID), lambda b: (b, 0, 0)),
            pl.BlockSpec((1, HIS, HID), lambda b: (b, 0, 0)),
            pl.BlockSpec((HID, HID), lambda b: (0, 0)),
            pl.BlockSpec((1, HID), lambda b: (0, 0)),
        ],
        out_specs=[
            pl.BlockSpec((1, CDD, K), lambda b: (b, 0, 0)),
            pl.BlockSpec((1, CDD, K), lambda b: (b, 0, 0)),
        ],
        out_shape=[
            jax.ShapeDtypeStruct((B, CDD, K), jnp.int32),
            jax.ShapeDtypeStruct((B, CDD, K), jnp.float32),
        ],
    )(cdd_repr, his_repr, sel_Wt, sel_b2)


NPAD = NW * RPW


def _gather_body(table, gspread, wspread, out, idx_v, w_v, rowbuf0, rowbuf1,
                 outbuf0, outbuf1, gsem0, gsem1, osem0, osem1):
    c = lax.axis_index("c")
    s = lax.axis_index("s")
    wid = s * NC + c
    pltpu.sync_copy(gspread.at[wid], idx_v)
    pltpu.sync_copy(wspread.at[wid], w_v)
    rowbufs = (rowbuf0, rowbuf1)
    outbufs = (outbuf0, outbuf1)
    gsems = (gsem0, gsem1)
    osems = (osem0, osem1)

    def start_gather(i):
        b = i % 2
        return pltpu.async_copy(table.at[idx_v.at[pl.ds(8 * i, 1)]],
                                rowbufs[b], gsems[b])

    gcopies = [None] * RPW
    ocopies = [None] * RPW
    gcopies[0] = start_gather(0)
    gcopies[1] = start_gather(1)
    for i in range(RPW):
        b = i % 2
        gcopies[i].wait()
        if i >= 2:
            ocopies[i - 2].wait()
        wvec = w_v[pl.ds(16 * i, 16)]
        rb, ob = rowbufs[b], outbufs[b]

        def mul(j, carry, rb=rb, ob=ob, wvec=wvec):
            sl = pl.ds(j * 16, 16)
            ob[0, sl] = rb[0, sl] * wvec
            return carry
        lax.fori_loop(0, CHUNKS, mul, 0, unroll=8)
        if i + 2 < RPW:
            gcopies[i + 2] = start_gather(i + 2)
        row = wid + i * NW
        ocopies[i] = pltpu.async_copy(ob, out.at[pl.ds(row, 1)], osems[b])
    ocopies[RPW - 2].wait()
    ocopies[RPW - 1].wait()


@functools.cache
def _gather():
    return pl.kernel(
        _gather_body,
        mesh=plsc.VectorSubcoreMesh(core_axis_name="c", subcore_axis_name="s"),
        out_type=jax.ShapeDtypeStruct((NPAD, ROW), jnp.float32),
        scratch_types=[
            pltpu.VMEM((RPW * 8,), jnp.int32),
            pltpu.VMEM((RPW * 16,), jnp.float32),
            pltpu.VMEM((1, ROW), jnp.float32),
            pltpu.VMEM((1, ROW), jnp.float32),
            pltpu.VMEM((1, ROW), jnp.float32),
            pltpu.VMEM((1, ROW), jnp.float32),
            pltpu.SemaphoreType.DMA,
            pltpu.SemaphoreType.DMA,
            pltpu.SemaphoreType.DMA,
            pltpu.SemaphoreType.DMA,
        ],
    )


def kernel(cdd_repr, his_repr, his_embedding, sel_W, sel_b):
    gidx, vals = _select(cdd_repr, his_repr, sel_W.T, sel_b.reshape(1, HID))
    table = his_embedding.reshape(B * HIS, ROW)
    gpad = jnp.pad(gidx.reshape(NROWS), (0, NPAD - NROWS))
    gmat = gpad.reshape(RPW, NW).T
    gspread = jnp.pad(gmat[:, :, None],
                      ((0, 0), (0, 0), (0, 7))).reshape(NW, RPW * 8)
    vpad = jnp.pad(vals.reshape(NROWS), (0, NPAD - NROWS))
    vmat = vpad.reshape(RPW, NW).T
    wspread = jnp.broadcast_to(vmat[:, :, None],
                               (NW, RPW, 16)).reshape(NW, RPW * 16)
    out = _gather()(table, gspread, wspread)
    return out[:NROWS].reshape(B, CDD, K, SIG, LVL, HID)


# speedup vs baseline: 1.0456x; 1.0456x over previous
import functools

import jax
import jax.numpy as jnp
from jax import lax
from jax.experimental import pallas as pl
from jax.experimental.pallas import tpu as pltpu
from jax.experimental.pallas import tpu_sc as plsc

B, CDD, HIS, K = 16, 5, 100, 5
SIG, LVL, HID = 32, 3, 256
THRESHOLD = 0.1
ROW = SIG * LVL * HID
NROWS = B * CDD * K
NC, NS = 2, 16
NW = NC * NS
CHUNKS = ROW // 16
RPW = -(-NROWS // NW)


def _select_body(cdd_ref, his_ref, wt_ref, b_ref, gidx_ref, vals_ref):
    bidx = pl.program_id(0)
    cdd = cdd_ref[0]
    his = his_ref[0]
    wt = wt_ref[...]
    bias = b_ref[...]
    dn = (((1,), (0,)), ((), ()))
    cp = lax.dot_general(cdd, wt, dn, preferred_element_type=jnp.float32) + bias
    hp = lax.dot_general(his, wt, dn, preferred_element_type=jnp.float32) + bias
    cn = jnp.sqrt(jnp.sum(cp * cp, axis=1, keepdims=True))
    hn = jnp.sqrt(jnp.sum(hp * hp, axis=1, keepdims=True))
    cp = cp / jnp.maximum(cn, 1e-12)
    hp = hp / jnp.maximum(hn, 1e-12)
    attn = lax.dot_general(cp, hp, (((1,), (1,)), ((), ())),
                           preferred_element_type=jnp.float32)
    cols = lax.broadcasted_iota(jnp.int32, (CDD, HIS), 1)
    a = attn
    vs, ids = [], []
    for _ in range(K):
        m = jnp.max(a, axis=1, keepdims=True)
        amax = jnp.min(jnp.where(a == m, cols, HIS), axis=1, keepdims=True)
        vs.append(m)
        ids.append(amax)
        a = jnp.where(cols == amax, -jnp.inf, a)
    vals = jnp.concatenate(vs, axis=1)
    idx = jnp.concatenate(ids, axis=1)
    masked = jnp.where(vals < THRESHOLD, 0.0, vals)
    gidx_ref[0] = idx + bidx * HIS
    vals_ref[0] = masked


def _select(cdd_repr, his_repr, sel_Wt, sel_b2):
    return pl.pallas_call(
        _select_body,
        grid=(B,),
        in_specs=[
            pl.BlockSpec((1, CDD, HID), lambda b: (b, 0, 0)),
            pl.BlockSpec((1, HIS, HID), lambda b: (b, 0, 0)),
            pl.BlockSpec((HID, HID), lambda b: (0, 0)),
            pl.BlockSpec((1, HID), lambda b: (0, 0)),
        ],
        out_specs=[
            pl.BlockSpec((1, CDD, K), lambda b: (b, 0, 0)),
            pl.BlockSpec((1, CDD, K), lambda b: (b, 0, 0)),
        ],
        out_shape=[
            jax.ShapeDtypeStruct((B, CDD, K), jnp.int32),
            jax.ShapeDtypeStruct((B, CDD, K), jnp.float32),
        ],
    )(cdd_repr, his_repr, sel_Wt, sel_b2)


NPAD = NW * RPW


def _gather_body(table, gspread, wspread, out, idx_v, w_v, rowbuf0, rowbuf1,
                 outbuf0, outbuf1, gsem0, gsem1, osem0, osem1):
    c = lax.axis_index("c")
    s = lax.axis_index("s")
    wid = s * NC + c
    pltpu.sync_copy(gspread.at[wid], idx_v)
    pltpu.sync_copy(wspread.at[wid], w_v)
    rowbufs = (rowbuf0, rowbuf1)
    outbufs = (outbuf0, outbuf1)
    gsems = (gsem0, gsem1)
    osems = (osem0, osem1)

    def start_gather(i):
        b = i % 2
        return pltpu.async_copy(table.at[idx_v.at[pl.ds(8 * i, 1)]],
                                rowbufs[b], gsems[b])

    gcopies = [None] * RPW
    ocopies = [None] * RPW
    gcopies[0] = start_gather(0)
    gcopies[1] = start_gather(1)
    for i in range(RPW - 1):
        b = i % 2
        gcopies[i].wait()
        if i >= 2:
            ocopies[i - 2].wait()
        wvec = w_v[pl.ds(16 * i, 16)]
        rb, ob = rowbufs[b], outbufs[b]

        def mul(j, carry, rb=rb, ob=ob, wvec=wvec):
            sl = pl.ds(j * 16, 16)
            ob[0, sl] = rb[0, sl] * wvec
            return carry
        lax.fori_loop(0, CHUNKS, mul, 0, unroll=8)
        if i + 2 < RPW:
            gcopies[i + 2] = start_gather(i + 2)
        row = wid + i * NW
        ocopies[i] = pltpu.async_copy(ob, out.at[pl.ds(row, 1)], osems[b])
    i = RPW - 1
    b = i % 2
    gcopies[i].wait()
    ocopies[i - 2].wait()
    wvec = w_v[pl.ds(16 * i, 16)]
    rb, ob = rowbufs[b], outbufs[b]

    def mul_last(j, carry, rb=rb, ob=ob, wvec=wvec):
        sl = pl.ds(j * 16, 16)
        ob[0, sl] = rb[0, sl] * wvec
        return carry
    lax.fori_loop(0, CHUNKS, mul_last, 0, unroll=8)

    @pl.when(wid + i * NW < NROWS)
    def _():
        pltpu.sync_copy(ob, out.at[pl.ds(wid + i * NW, 1)])
    ocopies[i - 1].wait()


@functools.cache
def _gather():
    return pl.kernel(
        _gather_body,
        mesh=plsc.VectorSubcoreMesh(core_axis_name="c", subcore_axis_name="s"),
        out_type=jax.ShapeDtypeStruct((NROWS, ROW), jnp.float32),
        scratch_types=[
            pltpu.VMEM((RPW * 8,), jnp.int32),
            pltpu.VMEM((RPW * 16,), jnp.float32),
            pltpu.VMEM((1, ROW), jnp.float32),
            pltpu.VMEM((1, ROW), jnp.float32),
            pltpu.VMEM((1, ROW), jnp.float32),
            pltpu.VMEM((1, ROW), jnp.float32),
            pltpu.SemaphoreType.DMA,
            pltpu.SemaphoreType.DMA,
            pltpu.SemaphoreType.DMA,
            pltpu.SemaphoreType.DMA,
        ],
    )


def kernel(cdd_repr, his_repr, his_embedding, sel_W, sel_b):
    gidx, vals = _select(cdd_repr, his_repr, sel_W.T, sel_b.reshape(1, HID))
    table = his_embedding.reshape(B * HIS, ROW)
    gpad = jnp.pad(gidx.reshape(NROWS), (0, NPAD - NROWS))
    gmat = gpad.reshape(RPW, NW).T
    gspread = jnp.pad(gmat[:, :, None],
                      ((0, 0), (0, 0), (0, 7))).reshape(NW, RPW * 8)
    vpad = jnp.pad(vals.reshape(NROWS), (0, NPAD - NROWS))
    vmat = vpad.reshape(RPW, NW).T
    wspread = jnp.broadcast_to(vmat[:, :, None],
                               (NW, RPW, 16)).reshape(NW, RPW * 16)
    out = _gather()(table, gspread, wspread)
    return out.reshape(B, CDD, K, SIG, LVL, HID)


# speedup vs baseline: 1.7498x; 1.6736x over previous
import functools

import jax
import jax.numpy as jnp
from jax import lax
from jax.experimental import pallas as pl
from jax.experimental.pallas import tpu as pltpu
from jax.experimental.pallas import tpu_sc as plsc

B, CDD, HIS, K = 16, 5, 100, 5
SIG, LVL, HID = 32, 3, 256
THRESHOLD = 0.1
ROW = SIG * LVL * HID
NROWS = B * CDD * K
NC, NS = 2, 16
NW = NC * NS
CHUNKS = ROW // 16
RPW = -(-NROWS // NW)
NPAD = NW * RPW


def _select_body(cdd_ref, his_ref, wt_ref, b_ref, gidx_ref, vals_ref):
    bidx = pl.program_id(0)
    cdd = cdd_ref[0]
    his = his_ref[0]
    wt = wt_ref[...]
    bias = b_ref[...]
    dn = (((1,), (0,)), ((), ()))
    cp = lax.dot_general(cdd, wt, dn, preferred_element_type=jnp.float32) + bias
    hp = lax.dot_general(his, wt, dn, preferred_element_type=jnp.float32) + bias
    cn = jnp.sqrt(jnp.sum(cp * cp, axis=1, keepdims=True))
    hn = jnp.sqrt(jnp.sum(hp * hp, axis=1, keepdims=True))
    cp = cp / jnp.maximum(cn, 1e-12)
    hp = hp / jnp.maximum(hn, 1e-12)
    attn = lax.dot_general(cp, hp, (((1,), (1,)), ((), ())),
                           preferred_element_type=jnp.float32)
    cols = lax.broadcasted_iota(jnp.int32, (CDD, HIS), 1)
    a = attn
    vs, ids = [], []
    for _ in range(K):
        m = jnp.max(a, axis=1, keepdims=True)
        amax = jnp.min(jnp.where(a == m, cols, HIS), axis=1, keepdims=True)
        vs.append(m)
        ids.append(amax)
        a = jnp.where(cols == amax, -jnp.inf, a)
    vals = jnp.concatenate(vs, axis=1)
    idx = jnp.concatenate(ids, axis=1)
    masked = jnp.where(vals < THRESHOLD, 0.0, vals)
    gidx_ref[0] = idx + bidx * HIS
    vals_ref[0] = masked


def _select(cdd_repr, his_repr, sel_Wt, sel_b2):
    return pl.pallas_call(
        _select_body,
        grid=(B,),
        in_specs=[
            pl.BlockSpec((1, CDD, HID), lambda b: (b, 0, 0)),
            pl.BlockSpec((1, HIS, HID), lambda b: (b, 0, 0)),
            pl.BlockSpec((HID, HID), lambda b: (0, 0)),
            pl.BlockSpec((1, HID), lambda b: (0, 0)),
        ],
        out_specs=[
            pl.BlockSpec((1, CDD, K), lambda b: (b, 0, 0)),
            pl.BlockSpec((1, CDD, K), lambda b: (b, 0, 0)),
        ],
        out_shape=[
            jax.ShapeDtypeStruct((B, CDD, K), jnp.int32),
            jax.ShapeDtypeStruct((B, CDD, K), jnp.float32),
        ],
    )(cdd_repr, his_repr, sel_Wt, sel_b2)


def _gather_body(table, gspread, wspread, out, idx_v, w_v, rowbuf0, rowbuf1,
                 outbuf0, outbuf1, gsem0, gsem1, osem0, osem1):
    c = lax.axis_index("c")
    s = lax.axis_index("s")
    wid = s * NC + c
    pltpu.sync_copy(gspread.at[wid], idx_v)
    pltpu.sync_copy(wspread.at[wid], w_v)
    rowbufs = (rowbuf0, rowbuf1)
    outbufs = (outbuf0, outbuf1)
    gsems = (gsem0, gsem1)
    osems = (osem0, osem1)

    def start_gather(i):
        b = i % 2
        return pltpu.async_copy(table.at[idx_v.at[pl.ds(8 * i, 1)]],
                                rowbufs[b], gsems[b])

    def scale(rb, ob, wvec):
        def mul(j, carry):
            sl = pl.ds(j * 16, 16)
            ob[0, sl] = rb[0, sl] * wvec
            return carry
        lax.fori_loop(0, CHUNKS, mul, 0, unroll=8)

    gcopies = [None] * RPW
    ocopies = [None] * RPW
    gcopies[0] = start_gather(0)
    gcopies[1] = start_gather(1)
    for i in range(RPW - 1):
        b = i % 2
        gcopies[i].wait()
        if i >= 2:
            ocopies[i - 2].wait()
        scale(rowbufs[b], outbufs[b], w_v[pl.ds(16 * i, 16)])
        if i + 2 < RPW:
            gcopies[i + 2] = start_gather(i + 2)
        row = wid + i * NW
        ocopies[i] = pltpu.async_copy(outbufs[b], out.at[pl.ds(row, 1)],
                                      osems[b])
    i = RPW - 1
    b = i % 2
    gcopies[i].wait()
    ocopies[i - 2].wait()
    scale(rowbufs[b], outbufs[b], w_v[pl.ds(16 * i, 16)])

    @pl.when(wid + i * NW < NROWS)
    def _():
        pltpu.sync_copy(outbufs[b], out.at[pl.ds(wid + i * NW, 1)])
    ocopies[i - 1].wait()


@functools.cache
def _gather():
    return pl.kernel(
        _gather_body,
        mesh=plsc.VectorSubcoreMesh(core_axis_name="c", subcore_axis_name="s"),
        out_type=jax.ShapeDtypeStruct((NROWS, ROW), jnp.float32),
        scratch_types=[
            pltpu.VMEM((RPW * 8,), jnp.int32),
            pltpu.VMEM((RPW * 16,), jnp.float32),
            pltpu.VMEM((1, ROW), jnp.float32),
            pltpu.VMEM((1, ROW), jnp.float32),
            pltpu.VMEM((1, ROW), jnp.float32),
            pltpu.VMEM((1, ROW), jnp.float32),
            pltpu.SemaphoreType.DMA,
            pltpu.SemaphoreType.DMA,
            pltpu.SemaphoreType.DMA,
            pltpu.SemaphoreType.DMA,
        ],
    )


def kernel(cdd_repr, his_repr, his_embedding, sel_W, sel_b):
    gidx, vals = _select(cdd_repr, his_repr, sel_W.T, sel_b.reshape(1, HID))
    table = his_embedding.transpose(0, 1, 3, 2, 4).reshape(B * HIS, ROW)
    gpad = jnp.pad(gidx.reshape(NROWS), (0, NPAD - NROWS))
    gmat = gpad.reshape(RPW, NW).T
    gspread = jnp.pad(gmat[:, :, None],
                      ((0, 0), (0, 0), (0, 7))).reshape(NW, RPW * 8)
    vpad = jnp.pad(vals.reshape(NROWS), (0, NPAD - NROWS))
    vmat = vpad.reshape(RPW, NW).T
    wspread = jnp.broadcast_to(vmat[:, :, None],
                               (NW, RPW, 16)).reshape(NW, RPW * 16)
    out = _gather()(table, gspread, wspread)
    return out.reshape(B, CDD, K, LVL, SIG, HID).transpose(0, 1, 2, 4, 3, 5)


# speedup vs baseline: 6.4211x; 3.6695x over previous
import functools

import jax
import jax.numpy as jnp
from jax import lax
from jax.experimental import pallas as pl
from jax.experimental.pallas import tpu as pltpu
from jax.experimental.pallas import tpu_sc as plsc

B, CDD, HIS, K = 16, 5, 100, 5
SIG, LVL, HID = 32, 3, 256
THRESHOLD = 0.1
ROW = SIG * LVL * HID
NROWS = B * CDD * K
NC, NS = 2, 16
NW = NC * NS
CHUNKS = ROW // 16
RPW = -(-NROWS // NW)
NPAD = NW * RPW


def _select_body(cdd_ref, his_ref, wt_ref, b_ref, gidx_ref, vals_ref):
    bidx = pl.program_id(0)
    cdd = cdd_ref[0]
    his = his_ref[0]
    wt = wt_ref[...]
    bias = b_ref[...]
    dn = (((1,), (0,)), ((), ()))
    cp = lax.dot_general(cdd, wt, dn, preferred_element_type=jnp.float32) + bias
    hp = lax.dot_general(his, wt, dn, preferred_element_type=jnp.float32) + bias
    cn = jnp.sqrt(jnp.sum(cp * cp, axis=1, keepdims=True))
    hn = jnp.sqrt(jnp.sum(hp * hp, axis=1, keepdims=True))
    cp = cp / jnp.maximum(cn, 1e-12)
    hp = hp / jnp.maximum(hn, 1e-12)
    attn = lax.dot_general(cp, hp, (((1,), (1,)), ((), ())),
                           preferred_element_type=jnp.float32)
    cols = lax.broadcasted_iota(jnp.int32, (CDD, HIS), 1)
    a = attn
    vs, ids = [], []
    for _ in range(K):
        m = jnp.max(a, axis=1, keepdims=True)
        amax = jnp.min(jnp.where(a == m, cols, HIS), axis=1, keepdims=True)
        vs.append(m)
        ids.append(amax)
        a = jnp.where(cols == amax, -jnp.inf, a)
    vals = jnp.concatenate(vs, axis=1)
    idx = jnp.concatenate(ids, axis=1)
    masked = jnp.where(vals < THRESHOLD, 0.0, vals)
    gidx_ref[0] = idx + bidx * HIS
    vals_ref[0] = masked


def _select(cdd_repr, his_repr, sel_Wt, sel_b2):
    return pl.pallas_call(
        _select_body,
        grid=(B,),
        in_specs=[
            pl.BlockSpec((1, CDD, HID), lambda b: (b, 0, 0)),
            pl.BlockSpec((1, HIS, HID), lambda b: (b, 0, 0)),
            pl.BlockSpec((HID, HID), lambda b: (0, 0)),
            pl.BlockSpec((1, HID), lambda b: (0, 0)),
        ],
        out_specs=[
            pl.BlockSpec((1, CDD, K), lambda b: (b, 0, 0)),
            pl.BlockSpec((1, CDD, K), lambda b: (b, 0, 0)),
        ],
        out_shape=[
            jax.ShapeDtypeStruct((B, CDD, K), jnp.int32),
            jax.ShapeDtypeStruct((B, CDD, K), jnp.float32),
        ],
    )(cdd_repr, his_repr, sel_Wt, sel_b2)


def _gather_body(table, gspread, wspread, out, idx_v, w_v, rowbuf0, rowbuf1,
                 outbuf0, outbuf1, gsem0, gsem1, osem0, osem1):
    c = lax.axis_index("c")
    s = lax.axis_index("s")
    wid = s * NC + c
    pltpu.sync_copy(gspread.at[wid], idx_v)
    pltpu.sync_copy(wspread.at[wid], w_v)
    rowbufs = (rowbuf0, rowbuf1)
    outbufs = (outbuf0, outbuf1)
    gsems = (gsem0, gsem1)
    osems = (osem0, osem1)

    def start_gather(i):
        b = i % 2
        return pltpu.async_copy(table.at[idx_v.at[pl.ds(8 * i, 1)]],
                                rowbufs[b], gsems[b])

    def scale(rb, ob, wvec):
        for lv in range(LVL):
            def mul(s, carry, lv=lv):
                for c in range(HID // 16):
                    sl = pl.ds(c * 16, 16)
                    ob[0, lv, s, sl] = rb[0, lv, s, sl] * wvec
                return carry
            lax.fori_loop(0, SIG, mul, 0)

    gcopies = [None] * RPW
    ocopies = [None] * RPW
    gcopies[0] = start_gather(0)
    gcopies[1] = start_gather(1)
    for i in range(RPW - 1):
        b = i % 2
        gcopies[i].wait()
        if i >= 2:
            ocopies[i - 2].wait()
        scale(rowbufs[b], outbufs[b], w_v[pl.ds(16 * i, 16)])
        if i + 2 < RPW:
            gcopies[i + 2] = start_gather(i + 2)
        row = wid + i * NW
        ocopies[i] = pltpu.async_copy(outbufs[b], out.at[pl.ds(row, 1)],
                                      osems[b])
    i = RPW - 1
    b = i % 2
    gcopies[i].wait()
    ocopies[i - 2].wait()
    scale(rowbufs[b], outbufs[b], w_v[pl.ds(16 * i, 16)])

    @pl.when(wid + i * NW < NROWS)
    def _():
        pltpu.sync_copy(outbufs[b], out.at[pl.ds(wid + i * NW, 1)])
    ocopies[i - 1].wait()


@functools.cache
def _gather():
    return pl.kernel(
        _gather_body,
        mesh=plsc.VectorSubcoreMesh(core_axis_name="c", subcore_axis_name="s"),
        out_type=jax.ShapeDtypeStruct((NROWS, LVL, SIG, HID), jnp.float32),
        scratch_types=[
            pltpu.VMEM((RPW * 8,), jnp.int32),
            pltpu.VMEM((RPW * 16,), jnp.float32),
            pltpu.VMEM((1, LVL, SIG, HID), jnp.float32),
            pltpu.VMEM((1, LVL, SIG, HID), jnp.float32),
            pltpu.VMEM((1, LVL, SIG, HID), jnp.float32),
            pltpu.VMEM((1, LVL, SIG, HID), jnp.float32),
            pltpu.SemaphoreType.DMA,
            pltpu.SemaphoreType.DMA,
            pltpu.SemaphoreType.DMA,
            pltpu.SemaphoreType.DMA,
        ],
    )


def kernel(cdd_repr, his_repr, his_embedding, sel_W, sel_b):
    gidx, vals = _select(cdd_repr, his_repr, sel_W.T, sel_b.reshape(1, HID))
    table = his_embedding.transpose(0, 1, 3, 2, 4).reshape(
        B * HIS, LVL, SIG, HID)
    gpad = jnp.pad(gidx.reshape(NROWS), (0, NPAD - NROWS))
    gmat = gpad.reshape(RPW, NW).T
    gspread = jnp.pad(gmat[:, :, None],
                      ((0, 0), (0, 0), (0, 7))).reshape(NW, RPW * 8)
    vpad = jnp.pad(vals.reshape(NROWS), (0, NPAD - NROWS))
    vmat = vpad.reshape(RPW, NW).T
    wspread = jnp.broadcast_to(vmat[:, :, None],
                               (NW, RPW, 16)).reshape(NW, RPW * 16)
    out = _gather()(table, gspread, wspread)
    return out.reshape(B, CDD, K, LVL, SIG, HID).transpose(0, 1, 2, 4, 3, 5)


# speedup vs baseline: 8.5607x; 1.3332x over previous
import functools

import jax
import jax.numpy as jnp
from jax import lax
from jax.experimental import pallas as pl
from jax.experimental.pallas import tpu as pltpu
from jax.experimental.pallas import tpu_sc as plsc

B, CDD, HIS, K = 16, 5, 100, 5
SIG, LVL, HID = 32, 3, 256
THRESHOLD = 0.1
ROW = SIG * LVL * HID
NROWS = B * CDD * K
NC, NS = 2, 16
NW = NC * NS
CHUNKS = ROW // 16
RPW = -(-NROWS // NW)
NPAD = NW * RPW


def _select_body(cdd_ref, his_ref, wt_ref, b_ref, gidx_ref, vals_ref):
    cdd = cdd_ref[...].reshape(B * CDD, HID)
    his = his_ref[...].reshape(B * HIS, HID)
    wt = wt_ref[...]
    bias = b_ref[...]
    dn = (((1,), (0,)), ((), ()))
    cp = lax.dot_general(cdd, wt, dn, preferred_element_type=jnp.float32) + bias
    hp = lax.dot_general(his, wt, dn, preferred_element_type=jnp.float32) + bias
    cn = jnp.sqrt(jnp.sum(cp * cp, axis=1, keepdims=True))
    hn = jnp.sqrt(jnp.sum(hp * hp, axis=1, keepdims=True))
    cp = cp / jnp.maximum(cn, 1e-12)
    hp = hp / jnp.maximum(hn, 1e-12)
    attn = lax.dot_general(cp, hp, (((1,), (1,)), ((), ())),
                           preferred_element_type=jnp.float32)
    rows = lax.broadcasted_iota(jnp.int32, (B * CDD, B * HIS), 0)
    cols = lax.broadcasted_iota(jnp.int32, (B * CDD, B * HIS), 1)
    a = jnp.where(rows // CDD == cols // HIS, attn, -jnp.inf)
    vs, ids = [], []
    for _ in range(K):
        m = jnp.max(a, axis=1, keepdims=True)
        amax = jnp.min(jnp.where(a == m, cols, B * HIS), axis=1, keepdims=True)
        vs.append(m)
        ids.append(amax)
        a = jnp.where(cols == amax, -jnp.inf, a)
    vals = jnp.concatenate(vs, axis=1)
    idx = jnp.concatenate(ids, axis=1)
    masked = jnp.where(vals < THRESHOLD, 0.0, vals)
    gidx_ref[...] = idx
    vals_ref[...] = masked


def _select(cdd_repr, his_repr, sel_Wt, sel_b2):
    return pl.pallas_call(
        _select_body,
        out_shape=[
            jax.ShapeDtypeStruct((B * CDD, K), jnp.int32),
            jax.ShapeDtypeStruct((B * CDD, K), jnp.float32),
        ],
    )(cdd_repr, his_repr, sel_Wt, sel_b2)


def _gather_body(table, gspread, wspread, out, idx_v, w_v, rowbuf0, rowbuf1,
                 outbuf0, outbuf1, gsem0, gsem1, osem0, osem1):
    c = lax.axis_index("c")
    s = lax.axis_index("s")
    wid = s * NC + c
    pltpu.sync_copy(gspread.at[wid], idx_v)
    pltpu.sync_copy(wspread.at[wid], w_v)
    rowbufs = (rowbuf0, rowbuf1)
    outbufs = (outbuf0, outbuf1)
    gsems = (gsem0, gsem1)
    osems = (osem0, osem1)

    def start_gather(i):
        b = i % 2
        return pltpu.async_copy(table.at[idx_v.at[pl.ds(8 * i, 1)]],
                                rowbufs[b], gsems[b])

    def scale(rb, ob, wvec):
        for lv in range(LVL):
            def mul(s, carry, lv=lv):
                for c in range(HID // 16):
                    sl = pl.ds(c * 16, 16)
                    ob[0, lv, s, sl] = rb[0, lv, s, sl] * wvec
                return carry
            lax.fori_loop(0, SIG, mul, 0)

    gcopies = [None] * RPW
    ocopies = [None] * RPW
    gcopies[0] = start_gather(0)
    gcopies[1] = start_gather(1)
    for i in range(RPW - 1):
        b = i % 2
        gcopies[i].wait()
        if i >= 2:
            ocopies[i - 2].wait()
        scale(rowbufs[b], outbufs[b], w_v[pl.ds(16 * i, 16)])
        if i + 2 < RPW:
            gcopies[i + 2] = start_gather(i + 2)
        row = wid + i * NW
        ocopies[i] = pltpu.async_copy(outbufs[b], out.at[pl.ds(row, 1)],
                                      osems[b])
    i = RPW - 1
    b = i % 2
    gcopies[i].wait()
    ocopies[i - 2].wait()
    scale(rowbufs[b], outbufs[b], w_v[pl.ds(16 * i, 16)])

    @pl.when(wid + i * NW < NROWS)
    def _():
        pltpu.sync_copy(outbufs[b], out.at[pl.ds(wid + i * NW, 1)])
    ocopies[i - 1].wait()


@functools.cache
def _gather():
    return pl.kernel(
        _gather_body,
        mesh=plsc.VectorSubcoreMesh(core_axis_name="c", subcore_axis_name="s"),
        out_type=jax.ShapeDtypeStruct((NROWS, LVL, SIG, HID), jnp.float32),
        scratch_types=[
            pltpu.VMEM((RPW * 8,), jnp.int32),
            pltpu.VMEM((RPW * 16,), jnp.float32),
            pltpu.VMEM((1, LVL, SIG, HID), jnp.float32),
            pltpu.VMEM((1, LVL, SIG, HID), jnp.float32),
            pltpu.VMEM((1, LVL, SIG, HID), jnp.float32),
            pltpu.VMEM((1, LVL, SIG, HID), jnp.float32),
            pltpu.SemaphoreType.DMA,
            pltpu.SemaphoreType.DMA,
            pltpu.SemaphoreType.DMA,
            pltpu.SemaphoreType.DMA,
        ],
    )


def kernel(cdd_repr, his_repr, his_embedding, sel_W, sel_b):
    gidx, vals = _select(cdd_repr, his_repr, sel_W.T, sel_b.reshape(1, HID))
    table = his_embedding.transpose(0, 1, 3, 2, 4).reshape(
        B * HIS, LVL, SIG, HID)
    gpad = jnp.pad(gidx.reshape(NROWS), (0, NPAD - NROWS))
    gmat = gpad.reshape(RPW, NW).T
    gspread = jnp.pad(gmat[:, :, None],
                      ((0, 0), (0, 0), (0, 7))).reshape(NW, RPW * 8)
    vpad = jnp.pad(vals.reshape(NROWS), (0, NPAD - NROWS))
    vmat = vpad.reshape(RPW, NW).T
    wspread = jnp.broadcast_to(vmat[:, :, None],
                               (NW, RPW, 16)).reshape(NW, RPW * 16)
    out = _gather()(table, gspread, wspread)
    return out.reshape(B, CDD, K, LVL, SIG, HID).transpose(0, 1, 2, 4, 3, 5)


# speedup vs baseline: 9.5816x; 1.1193x over previous
import functools

import jax
import jax.numpy as jnp
from jax import lax
from jax.experimental import pallas as pl
from jax.experimental.pallas import tpu as pltpu
from jax.experimental.pallas import tpu_sc as plsc

B, CDD, HIS, K = 16, 5, 100, 5
SIG, LVL, HID = 32, 3, 256
THRESHOLD = 0.1
ROW = SIG * LVL * HID
NROWS = B * CDD * K
NC, NS = 2, 16
NW = NC * NS
CHUNKS = ROW // 16
RPW = -(-NROWS // NW)
NPAD = NW * RPW


def _select_body(cdd_ref, his_ref, w_ref, b_ref, gidx_ref, vals_ref):
    cdd = cdd_ref[...].reshape(CDD * B, HID)
    his = his_ref[...].reshape(HIS * B, HID)
    w = w_ref[...]
    bias = b_ref[...]
    dn = (((1,), (1,)), ((), ()))
    cp = lax.dot_general(cdd, w, dn, preferred_element_type=jnp.float32) + bias
    hp = lax.dot_general(his, w, dn, preferred_element_type=jnp.float32) + bias
    cn = jnp.sqrt(jnp.sum(cp * cp, axis=1, keepdims=True))
    hn = jnp.sqrt(jnp.sum(hp * hp, axis=1, keepdims=True))
    cp = cp / jnp.maximum(cn, 1e-12)
    hp = hp / jnp.maximum(hn, 1e-12)
    attn = lax.dot_general(cp, hp, (((1,), (1,)), ((), ())),
                           preferred_element_type=jnp.float32)
    rows = lax.broadcasted_iota(jnp.int32, (CDD * B, HIS * B), 0)
    cols = lax.broadcasted_iota(jnp.int32, (CDD * B, HIS * B), 1)
    a = jnp.where(rows % B == cols % B, attn, -jnp.inf)
    vs, ids = [], []
    for _ in range(K):
        m = jnp.max(a, axis=1, keepdims=True)
        amax = jnp.min(jnp.where(a == m, cols, HIS * B), axis=1, keepdims=True)
        vs.append(m)
        ids.append(amax)
        a = jnp.where(cols == amax, -jnp.inf, a)
    vals = jnp.concatenate(vs, axis=1)
    idx = jnp.concatenate(ids, axis=1)
    masked = jnp.where(vals < THRESHOLD, 0.0, vals)
    gidx_ref[...] = (idx % B) * HIS + idx // B
    vals_ref[...] = masked


def _select(cdd_t, his_t, sel_W, sel_b2):
    return pl.pallas_call(
        _select_body,
        out_shape=[
            jax.ShapeDtypeStruct((CDD * B, K), jnp.int32),
            jax.ShapeDtypeStruct((CDD * B, K), jnp.float32),
        ],
    )(cdd_t, his_t, sel_W, sel_b2)


def _gather_body(table, gspread, wspread, out, idx_v, w_v, rowbuf0, rowbuf1,
                 outbuf0, outbuf1, gsem0, gsem1, osem0, osem1):
    c = lax.axis_index("c")
    s = lax.axis_index("s")
    wid = s * NC + c
    pltpu.sync_copy(gspread.at[wid], idx_v)
    pltpu.sync_copy(wspread.at[wid], w_v)
    rowbufs = (rowbuf0, rowbuf1)
    outbufs = (outbuf0, outbuf1)
    gsems = (gsem0, gsem1)
    osems = (osem0, osem1)

    def start_gather(i):
        b = i % 2
        return pltpu.async_copy(table.at[idx_v.at[pl.ds(8 * i, 1)]],
                                rowbufs[b], gsems[b])

    def scale(rb, ob, wvec):
        for lv in range(LVL):
            def mul(s, carry, lv=lv):
                for c in range(HID // 16):
                    sl = pl.ds(c * 16, 16)
                    ob[0, lv, s, sl] = rb[0, lv, s, sl] * wvec
                return carry
            lax.fori_loop(0, SIG, mul, 0)

    gcopies = [None] * RPW
    ocopies = [None] * RPW
    gcopies[0] = start_gather(0)
    gcopies[1] = start_gather(1)
    for i in range(RPW - 1):
        b = i % 2
        gcopies[i].wait()
        if i >= 2:
            ocopies[i - 2].wait()
        scale(rowbufs[b], outbufs[b], w_v[pl.ds(16 * i, 16)])
        if i + 2 < RPW:
            gcopies[i + 2] = start_gather(i + 2)
        row = wid + i * NW
        ocopies[i] = pltpu.async_copy(outbufs[b], out.at[pl.ds(row, 1)],
                                      osems[b])
    i = RPW - 1
    b = i % 2
    gcopies[i].wait()
    ocopies[i - 2].wait()
    scale(rowbufs[b], outbufs[b], w_v[pl.ds(16 * i, 16)])

    @pl.when(wid + i * NW < NROWS)
    def _():
        pltpu.sync_copy(outbufs[b], out.at[pl.ds(wid + i * NW, 1)])
    ocopies[i - 1].wait()


@functools.cache
def _gather():
    return pl.kernel(
        _gather_body,
        mesh=plsc.VectorSubcoreMesh(core_axis_name="c", subcore_axis_name="s"),
        out_type=jax.ShapeDtypeStruct((NROWS, LVL, SIG, HID), jnp.float32),
        scratch_types=[
            pltpu.VMEM((RPW * 8,), jnp.int32),
            pltpu.VMEM((RPW * 16,), jnp.float32),
            pltpu.VMEM((1, LVL, SIG, HID), jnp.float32),
            pltpu.VMEM((1, LVL, SIG, HID), jnp.float32),
            pltpu.VMEM((1, LVL, SIG, HID), jnp.float32),
            pltpu.VMEM((1, LVL, SIG, HID), jnp.float32),
            pltpu.SemaphoreType.DMA,
            pltpu.SemaphoreType.DMA,
            pltpu.SemaphoreType.DMA,
            pltpu.SemaphoreType.DMA,
        ],
    )


def kernel(cdd_repr, his_repr, his_embedding, sel_W, sel_b):
    gidx80, vals80 = _select(cdd_repr.transpose(1, 0, 2),
                             his_repr.transpose(1, 0, 2),
                             sel_W, sel_b.reshape(1, HID))
    gidx = gidx80.reshape(CDD, B, K).transpose(1, 0, 2)
    vals = vals80.reshape(CDD, B, K).transpose(1, 0, 2)
    table = his_embedding.transpose(0, 1, 3, 2, 4).reshape(
        B * HIS, LVL, SIG, HID)
    gpad = jnp.pad(gidx.reshape(NROWS), (0, NPAD - NROWS))
    gmat = gpad.reshape(RPW, NW).T
    gspread = jnp.pad(gmat[:, :, None],
                      ((0, 0), (0, 0), (0, 7))).reshape(NW, RPW * 8)
    vpad = jnp.pad(vals.reshape(NROWS), (0, NPAD - NROWS))
    vmat = vpad.reshape(RPW, NW).T
    wspread = jnp.broadcast_to(vmat[:, :, None],
                               (NW, RPW, 16)).reshape(NW, RPW * 16)
    out = _gather()(table, gspread, wspread)
    return out.reshape(B, CDD, K, LVL, SIG, HID).transpose(0, 1, 2, 4, 3, 5)
